# Initial kernel scaffold; baseline (speedup 1.0000x reference)
import jax
import jax.numpy as jnp
from jax.experimental import pallas as pl
from jax.experimental.pallas import tpu as pltpu
from jax.experimental.pallas import tpu_sc as plsc

DIM = 64
WINDOW = 512


def kernel(x, table):
    batch, seq = x.shape
    num_idx = batch * seq
    idx = x.reshape(1, num_idx)

    mesh = plsc.VectorSubcoreMesh(core_axis_name="core",
                                  subcore_axis_name="subcore")

    @pl.kernel(
        out_type=jax.ShapeDtypeStruct((num_idx, DIM), table.dtype),
        mesh=mesh,
    )
    def gather_kernel(table_hbm, idx_hbm, out_hbm):
        def body(idx_vmem, out_vmem):
            pltpu.sync_copy(table_hbm.at[idx_vmem.at[0]], out_vmem)

        pltpu.emit_pipeline(
            body,
            grid=(num_idx // WINDOW,),
            in_specs=[pl.BlockSpec((1, WINDOW), lambda i: (0, i))],
            out_specs=[pl.BlockSpec((WINDOW, DIM), lambda i: (i, 0))],
            core_axis_name=("core", "subcore"),
            dimension_semantics=(pltpu.PARALLEL,),
        )(idx_hbm, out_hbm)

    out = gather_kernel(table, idx)
    return out.reshape(batch, seq, DIM)


# speedup vs baseline: 1.9444x; 1.9444x over previous
import jax
import jax.numpy as jnp
from jax import lax
from jax.experimental import pallas as pl
from jax.experimental.layout import Format, Layout, with_layout_constraint
from jax.experimental.pallas import tpu as pltpu
from jax.experimental.pallas import tpu_sc as plsc

DIM = 64
NUM_TILES = 32
CHUNK = 512


def kernel(x, table):
    batch, seq = x.shape
    num_idx = batch * seq
    per_tile = num_idx // NUM_TILES
    steps = per_tile // CHUNK
    idx = x.reshape(num_idx)

    table = with_layout_constraint(
        table, Layout(major_to_minor=(0, 1), tiling=((16,),)))

    mesh = plsc.VectorSubcoreMesh(core_axis_name="c", subcore_axis_name="s")

    @pl.kernel(
        out_type=jax.ShapeDtypeStruct((num_idx, DIM), table.dtype),
        mesh=mesh,
        scratch_types=[
            pltpu.VMEM((CHUNK,), jnp.int32),
            pltpu.VMEM((CHUNK, DIM), jnp.float32),
            pltpu.SemaphoreType.DMA,
        ],
    )
    def gather_kernel(table_hbm, idx_hbm, out_hbm, idx_v, rows_v, sem):
        wid = lax.axis_index("s") * 2 + lax.axis_index("c")
        tile_base = wid * per_tile

        @pl.loop(0, steps)
        def _(i):
            base = tile_base + i * CHUNK
            pltpu.sync_copy(idx_hbm.at[pl.ds(base, CHUNK)], idx_v)
            pltpu.async_copy(table_hbm.at[idx_v], rows_v, sem).wait()
            pltpu.sync_copy(rows_v, out_hbm.at[pl.ds(base, CHUNK)])

    out = gather_kernel(table, idx)
    return out.reshape(batch, seq, DIM)


# speedup vs baseline: 2.3217x; 1.1940x over previous
import jax
import jax.numpy as jnp
from jax import lax
from jax.experimental import pallas as pl
from jax.experimental.layout import Layout, with_layout_constraint
from jax.experimental.pallas import tpu as pltpu
from jax.experimental.pallas import tpu_sc as plsc

DIM = 64
NUM_TILES = 32
CHUNK = 800


def kernel(x, table):
    batch, seq = x.shape
    num_idx = batch * seq
    per_tile = num_idx // NUM_TILES
    steps = per_tile // CHUNK
    idx = x.reshape(num_idx)

    table = with_layout_constraint(
        table, Layout(major_to_minor=(0, 1), tiling=((16,),)))

    mesh = plsc.VectorSubcoreMesh(core_axis_name="c", subcore_axis_name="s")

    @pl.kernel(
        out_type=jax.ShapeDtypeStruct((num_idx, DIM), table.dtype),
        mesh=mesh,
        scratch_types=[
            pltpu.VMEM((CHUNK,), jnp.int32),
            pltpu.VMEM((CHUNK, DIM), jnp.float32),
            pltpu.SemaphoreType.DMA,
        ],
    )
    def gather_kernel(table_hbm, idx_hbm, out_hbm, idx_v, rows_v, sem):
        wid = lax.axis_index("s") * 2 + lax.axis_index("c")
        tile_base = wid * per_tile

        @pl.loop(0, steps)
        def _(i):
            base = tile_base + i * CHUNK
            pltpu.sync_copy(idx_hbm.at[pl.ds(base, CHUNK)], idx_v)
            pltpu.async_copy(table_hbm.at[idx_v], rows_v, sem).wait()
            pltpu.sync_copy(rows_v, out_hbm.at[pl.ds(base, CHUNK)])

    out = gather_kernel(table, idx)
    out = with_layout_constraint(
        out, Layout(major_to_minor=(0, 1), tiling=((8, 128),)))
    out = out.reshape(batch, seq, DIM)
    return with_layout_constraint(
        out, Layout(major_to_minor=(0, 1, 2), tiling=((8, 128),)))
